# baseline (speedup 1.0000x reference)
import jax
import jax.numpy as jnp
from jax import lax
from jax.experimental import pallas as pl
from jax.experimental.pallas import tpu as pltpu
from jax.experimental.pallas import tpu_sc as plsc

_VHS = 100
_NVT = 3
_W = 32
_BN = 512
_NTILES = 16
_NCORES = 2


_GI_DIMS = (((0,), (0,)), ((), ()))


def _sig(v):
    return 0.5 * jnp.tanh(0.5 * v) + 0.5


def _pre_body(xat_ref, a8_ref, bhh_ref, wgT_ref, bg_ref, wmT_ref, m0):
    gi = jax.lax.dot_general(xat_ref[...], a8_ref[...], _GI_DIMS,
                             preferred_element_type=jnp.float32)
    bhh = bhh_ref[...]
    r = _sig(gi[:, :128] + bhh[:, :128])
    z = _sig(gi[:, 128:256] + bhh[:, 128:256])
    n = jnp.tanh(gi[:, 256:] + r * bhh[:, 256:])
    h1 = (1.0 - z) * n
    g = _sig(jnp.dot(h1, wgT_ref[...], preferred_element_type=jnp.float32)
             + bg_ref[...])
    p = jnp.dot(h1, wmT_ref[...], preferred_element_type=jnp.float32)
    m0[...] = g * p


def _post_body(xat_ref, agg_ref, a8_ref, bhh_ref, whhT_ref, wpT_ref, bp_ref,
               out_ref):
    gi = jax.lax.dot_general(xat_ref[...], a8_ref[...], _GI_DIMS,
                             preferred_element_type=jnp.float32)
    agg128 = agg_ref[...]
    gh = jnp.dot(agg128, whhT_ref[...],
                 preferred_element_type=jnp.float32) + bhh_ref[...]
    r = _sig(gi[:, :128] + gh[:, :128])
    z = _sig(gi[:, 128:256] + gh[:, 128:256])
    n = jnp.tanh(gi[:, 256:] + r * gh[:, 256:])
    h2 = (1.0 - z) * n + z * agg128
    out_ref[...] = (jnp.dot(h2, wpT_ref[...],
                            preferred_element_type=jnp.float32)
                    + bp_ref[...])


_DEPTH = 4


def _make_sc_body(NP, E, EPT, BATCH):
    NBATCH = EPT // BATCH
    ZROWS = NP // _NTILES
    nz_full, nz_rem = divmod(ZROWS, BATCH)
    KMAX = (NBATCH + _DEPTH + 3) // 4

    def body(edge_hbm, srcx_hbm, m_hbm, o_hbm,
             sv0, sv1, sv2, sv3, dv0, dv1, dv2, dv3,
             rw0, rw1, rw2, rw3, accum, *sems):
        c = lax.axis_index("c")
        s = lax.axis_index("s")
        base_z = s * ZROWS
        base_e = s * EPT
        SV = (sv0, sv1, sv2, sv3)
        DV = (dv0, dv1, dv2, dv3)
        RW = (rw0, rw1, rw2, rw3)
        IS = sems[0:4]
        ID = sems[4:8]
        GS = sems[8:12]
        SS = sems[12:16]

        def zero_buf(buf):
            def zstore(i, carry):
                z16 = jnp.zeros((16,), jnp.float32)
                buf[i, pl.ds(0, 16)] = z16
                buf[i, pl.ds(16, 16)] = z16
                return carry
            lax.fori_loop(0, BATCH, zstore, 0)

        def run_chunk(cid):
            zero_buf(rw0)
            for k in range(nz_full):
                pltpu.sync_copy(rw0,
                                accum.at[pl.ds(base_z + k * BATCH, BATCH)])
            if nz_rem:
                pltpu.sync_copy(
                    rw0.at[pl.ds(0, nz_rem)],
                    accum.at[pl.ds(base_z + nz_full * BATCH, nz_rem)])
            plsc.subcore_barrier()

            def idx_start(t, b):
                off = base_e + b * BATCH
                pltpu.async_copy(
                    srcx_hbm.at[pl.ds(cid * E + off, BATCH)], SV[t], IS[t])
                pltpu.async_copy(
                    edge_hbm.at[1, pl.ds(off, BATCH)], DV[t], ID[t])

            def idx_wait(t):
                pltpu.make_async_copy(
                    srcx_hbm.at[pl.ds(base_e, BATCH)], SV[t], IS[t]).wait()
                pltpu.make_async_copy(
                    edge_hbm.at[1, pl.ds(base_e, BATCH)], DV[t], ID[t]).wait()

            def gather_start(t):
                pltpu.async_copy(m_hbm.at[SV[t]], RW[t], GS[t])

            def gather_wait(t):
                pltpu.make_async_copy(m_hbm.at[SV[t]], RW[t], GS[t]).wait()

            def scatter_start(t):
                pltpu.async_copy(RW[t], accum.at[DV[t]], SS[t], add=True)

            def scatter_wait(t):
                pltpu.make_async_copy(RW[t], accum.at[DV[t]], SS[t]).wait()

            def kbody(k, carry):
                for j in range(4):
                    b = 4 * k + j

                    @pl.when((b >= _DEPTH) & (b < NBATCH + _DEPTH))
                    def _():
                        scatter_wait(j)

                    @pl.when(b < NBATCH)
                    def _():
                        idx_start(j, b)

                    bg = b - 2
                    jg = (j - 2) % 4

                    @pl.when((bg >= 0) & (bg < NBATCH))
                    def _():
                        idx_wait(jg)
                        gather_start(jg)

                    bs = b - 3
                    js = (j - 3) % 4

                    @pl.when((bs >= 0) & (bs < NBATCH))
                    def _():
                        gather_wait(js)
                        scatter_start(js)
                return carry
            lax.fori_loop(0, KMAX, kbody, 0)

            plsc.subcore_barrier()
            pltpu.sync_copy(accum.at[pl.ds(base_z, ZROWS)],
                            o_hbm.at[pl.ds(base_z, ZROWS),
                                     pl.ds(cid * _W, _W)])
            plsc.subcore_barrier()

        @pl.when(c == 0)
        def _():
            run_chunk(0)
            run_chunk(1)

        @pl.when(c == 1)
        def _():
            run_chunk(2)
            run_chunk(3)

    return body


def kernel(x, edge_index, W_ih, b_ih, W_hh, b_hh, Wg, bg, Wm, Wp, bp):
    N = x.shape[0]
    E = edge_index.shape[1]
    NP = -(-N // _BN) * _BN
    grid = NP // _BN
    EPT = E // _NTILES
    BATCH = 200
    while EPT % BATCH or BATCH % 8:
        BATCH -= 8

    xa = jnp.concatenate([x, jnp.ones((N, 1), jnp.float32)], axis=1)
    xat = jnp.pad(xa.T, ((0, 4), (0, NP - N)))

    def _group384(w):
        return jnp.pad(w.reshape(w.shape[:-1] + (3, _VHS)),
                       [(0, 0)] * (w.ndim - 1) + [(0, 0), (0, 28)]
                       ).reshape(w.shape[:-1] + (384,))

    A8 = _group384(jnp.pad(
        jnp.concatenate([W_ih.T, b_ih[None, :]], axis=0),
        ((0, 4), (0, 0))))
    bhh2 = _group384(b_hh[None, :])
    wgT = jnp.pad(Wg.T, ((0, 28), (0, 28)))
    wmT = jnp.pad(Wm.T, ((0, 28), (0, 28)))
    bg2 = jnp.pad(bg, (0, 28))[None, :]

    m = pl.pallas_call(
        _pre_body,
        grid=(grid,),
        in_specs=[
            pl.BlockSpec((8, _BN), lambda i: (0, i)),
            pl.BlockSpec((8, 384), lambda i: (0, 0)),
            pl.BlockSpec((1, 384), lambda i: (0, 0)),
            pl.BlockSpec((128, 128), lambda i: (0, 0)),
            pl.BlockSpec((1, 128), lambda i: (0, 0)),
            pl.BlockSpec((128, 128), lambda i: (0, 0)),
        ],
        out_specs=pl.BlockSpec((_BN, 128), lambda i: (i, 0)),
        out_shape=jax.ShapeDtypeStruct((NP, 128), jnp.float32),
    )(xat, A8, bhh2, wgT, bg2, wmT)
    mview = jnp.reshape(m, (4 * NP, _W))

    srcx = ((edge_index[0] * 4)[None, :]
            + jnp.arange(4, dtype=jnp.int32)[:, None]).reshape(4 * E)
    mesh = plsc.VectorSubcoreMesh(core_axis_name="c", subcore_axis_name="s",
                                  num_cores=_NCORES, num_subcores=_NTILES)
    sc_fn = pl.kernel(
        _make_sc_body(NP, E, EPT, BATCH),
        out_type=jax.ShapeDtypeStruct((NP, 128), jnp.float32),
        mesh=mesh,
        compiler_params=pltpu.CompilerParams(use_tc_tiling_on_sc=False),
        scratch_types=(
            [pltpu.VMEM((BATCH,), jnp.int32)] * 8
            + [pltpu.VMEM((BATCH, _W), jnp.float32)] * 4
            + [pltpu.VMEM_SHARED((NP, _W), jnp.float32)]
            + [pltpu.SemaphoreType.DMA] * 16),
    )
    agg = sc_fn(edge_index, srcx, mview)

    whhT = _group384(jnp.pad(W_hh.T, ((0, 128 - _VHS), (0, 0))))
    wpT = jnp.pad(Wp.T, ((0, 128 - _VHS), (0, 0)))
    out = pl.pallas_call(
        _post_body,
        grid=(grid,),
        in_specs=[
            pl.BlockSpec((8, _BN), lambda i: (0, i)),
            pl.BlockSpec((_BN, 128), lambda i: (i, 0)),
            pl.BlockSpec((8, 384), lambda i: (0, 0)),
            pl.BlockSpec((1, 384), lambda i: (0, 0)),
            pl.BlockSpec((128, 384), lambda i: (0, 0)),
            pl.BlockSpec((128, _NVT), lambda i: (0, 0)),
            pl.BlockSpec((1, _NVT), lambda i: (0, 0)),
        ],
        out_specs=pl.BlockSpec((_BN, _NVT), lambda i: (i, 0)),
        out_shape=jax.ShapeDtypeStruct((N, _NVT), jnp.float32),
    )(xat, agg, A8, bhh2, whhT, wpT, bp[None, :])
    return out
